# baseline (speedup 1.0000x reference)
import functools
import math
import jax
import jax.numpy as jnp
from jax import lax
from jax.experimental import pallas as pl
from jax.experimental.pallas import tpu as pltpu
from jax.experimental.pallas import tpu_sc as plsc

_NC = 2
_NS = 16
_NW = _NC * _NS
_EB = 64
_RING1 = 2
_RING2 = 4

_MESH = plsc.VectorSubcoreMesh(
    core_axis_name="c", subcore_axis_name="s", num_cores=_NC, num_subcores=_NS
)


def _pad_edges(src, dst, n):
    e = src.shape[0]
    r = max(_RING1, _RING2, 8)
    nb = -(-(-(-e // (_NW * _EB))) // r) * r
    e_pad = _NW * nb * _EB
    pad = e_pad - e
    if pad:
        j = jnp.arange(pad, dtype=jnp.int32)
        src = jnp.concatenate([src, j % 16])
        dst = jnp.concatenate([dst, n + (j % 8)])
    return src.reshape(_NW, nb, _EB), dst.reshape(_NW, nb, _EB), nb


def _make_deg_kernel(n, nb, rpt):
    n_pad = _NS * rpt
    last = n - (_NS - 1) * rpt

    @functools.partial(
        pl.kernel,
        out_type=[
            jax.ShapeDtypeStruct((n,), jnp.float32),
            jax.ShapeDtypeStruct((n,), jnp.float32),
        ],
        mesh=_MESH,
        scratch_types=[
            pltpu.VMEM((nb, _EB), jnp.int32),
            pltpu.VMEM((_EB,), jnp.float32),
            pltpu.VMEM((16,), jnp.float32),
            pltpu.VMEM((rpt,), jnp.float32),
            pltpu.VMEM_SHARED((n_pad,), jnp.float32),
            pltpu.SemaphoreType.DMA,
        ],
    )
    def deg_kernel(dst_hbm, out0_hbm, out1_hbm, dst_v, ones_v, z_v, wb_v,
                   hist_sh, sem):
        c = lax.axis_index("c")
        s = lax.axis_index("s")
        w = s * _NC + c
        start = pl.multiple_of(s * rpt, rpt)

        z_v[...] = jnp.zeros((16,), jnp.float32)
        for i in range(_EB // 16):
            ones_v[pl.ds(i * 16, 16)] = jnp.ones((16,), jnp.float32)
        for k in range(rpt // 16):
            pltpu.sync_copy(z_v, hist_sh.at[pl.ds(start + k * 16, 16)])
        plsc.subcore_barrier()

        pltpu.sync_copy(dst_hbm.at[w], dst_v)

        def step(j, carry):
            pltpu.sync_copy(ones_v, hist_sh.at[dst_v.at[j]], add=True)
            return carry

        lax.fori_loop(0, nb, step, 0)
        plsc.subcore_barrier()

        for cc, out_hbm in ((0, out0_hbm), (1, out1_hbm)):

            @pl.when(jnp.logical_and(c == cc, s < _NS - 1))
            def _():
                pltpu.sync_copy(hist_sh.at[pl.ds(start, rpt)], wb_v)
                pltpu.sync_copy(wb_v, out_hbm.at[pl.ds(start, rpt)])

            @pl.when(jnp.logical_and(c == cc, s == _NS - 1))
            def _():
                pltpu.sync_copy(
                    hist_sh.at[pl.ds((_NS - 1) * rpt, last)], wb_v.at[pl.ds(0, last)]
                )
                pltpu.sync_copy(
                    wb_v.at[pl.ds(0, last)], out_hbm.at[pl.ds((_NS - 1) * rpt, last)]
                )

    return deg_kernel


def _agg_cw(n, rpt, d):
    cw = math.gcd(rpt, n - (_NS - 1) * rpt)
    while cw * d * 4 > 20 * 1024 and cw % 16 == 0:
        cw //= 2
    return cw


def _make_agg_kernel(n, nb, ring, rpt, d):
    n_pad = _NS * rpt
    last = n - (_NS - 1) * rpt
    cw = _agg_cw(n, rpt, d)

    @functools.partial(
        pl.kernel,
        out_type=jax.ShapeDtypeStruct((_NC, n, d), jnp.float32),
        mesh=_MESH,
        compiler_params=pltpu.CompilerParams(use_tc_tiling_on_sc=False),
        scratch_types=[
            pltpu.VMEM((nb, _EB), jnp.int32),
            pltpu.VMEM((nb, _EB), jnp.int32),
            [pltpu.VMEM((_EB, d), jnp.float32)] * ring,
            pltpu.VMEM((cw, d), jnp.float32),
            pltpu.VMEM_SHARED((n_pad, d), jnp.float32),
            [pltpu.SemaphoreType.DMA] * ring,
        ],
    )
    def agg_kernel(rows_hbm, src_hbm, dst_hbm, z_hbm, out_hbm,
                   src_v, dst_v, msgs, wb_v, acc_sh, sems):
        c = lax.axis_index("c")
        s = lax.axis_index("s")
        w = s * _NC + c
        start = pl.multiple_of(s * rpt, rpt)

        pltpu.sync_copy(z_hbm, wb_v)
        for k in range(rpt // cw):
            pltpu.sync_copy(wb_v, acc_sh.at[pl.ds(start + k * cw, cw)])
        plsc.subcore_barrier()

        pltpu.sync_copy(src_hbm.at[w], src_v)
        pltpu.sync_copy(dst_hbm.at[w], dst_v)

        for b in range(ring - 1):
            pltpu.async_copy(rows_hbm.at[src_v.at[b]], msgs[b], sems[b])

        def step(i, carry):
            j = i * ring
            for b in range(ring):
                jj = j + b
                bn = (b + ring - 1) % ring

                @pl.when(jj + ring - 1 < nb)
                def _():
                    pltpu.async_copy(
                        rows_hbm.at[src_v.at[jj + ring - 1]], msgs[bn], sems[bn]
                    )

                pltpu.make_async_copy(
                    rows_hbm.at[src_v.at[jj]], msgs[b], sems[b]
                ).wait()
                pltpu.sync_copy(msgs[b], acc_sh.at[dst_v.at[jj]], add=True)
            return carry

        lax.fori_loop(0, nb // ring, step, 0)
        plsc.subcore_barrier()

        @pl.when(s < _NS - 1)
        def _():
            for t in range(rpt // cw):
                pltpu.sync_copy(acc_sh.at[pl.ds(start + t * cw, cw)], wb_v)
                pltpu.sync_copy(wb_v, out_hbm.at[c, pl.ds(start + t * cw, cw)])

        @pl.when(s == _NS - 1)
        def _():
            for t in range(last // cw):
                off = (_NS - 1) * rpt + t * cw
                pltpu.sync_copy(acc_sh.at[pl.ds(off, cw)], wb_v)
                pltpu.sync_copy(wb_v, out_hbm.at[c, pl.ds(off, cw)])

    return agg_kernel


def _tc_matmul(x, w1, bn):
    n, d_in = x.shape
    d_hid = w1.shape[1]

    def body(x_ref, w_ref, t_ref):
        t_ref[...] = jnp.dot(
            x_ref[...], w_ref[...], preferred_element_type=jnp.float32
        )

    return pl.pallas_call(
        body,
        grid=(n // bn,),
        in_specs=[
            pl.BlockSpec((bn, d_in), lambda i: (i, 0)),
            pl.BlockSpec((d_in, d_hid), lambda i: (0, 0)),
        ],
        out_specs=pl.BlockSpec((bn, d_hid), lambda i: (i, 0)),
        out_shape=jax.ShapeDtypeStruct((n, d_hid), jnp.float32),
    )(x, w1)


def _tc_scale(dp2, t, bn):
    n, d_hid = t.shape

    def body(dp_ref, t_ref, xs_ref, dinv_ref):
        deg = dp_ref[0] + dp_ref[1] + 1.0
        dinv = lax.rsqrt(jnp.maximum(deg, 1.0))
        xs_ref[...] = t_ref[...] * dinv
        dinv_ref[...] = dinv

    return pl.pallas_call(
        body,
        grid=(n // bn,),
        in_specs=[
            pl.BlockSpec((_NC, bn, 1), lambda i: (0, i, 0)),
            pl.BlockSpec((bn, d_hid), lambda i: (i, 0)),
        ],
        out_specs=[
            pl.BlockSpec((bn, d_hid), lambda i: (i, 0)),
            pl.BlockSpec((bn, 1), lambda i: (i, 0)),
        ],
        out_shape=[
            jax.ShapeDtypeStruct((n, d_hid), jnp.float32),
            jax.ShapeDtypeStruct((n, 1), jnp.float32),
        ],
    )(dp2, t)


def _tc_mid(p1, xs, dinv, b1, w2, bn):
    n, d_hid = xs.shape
    d_out = w2.shape[1]

    def body(p_ref, xs_ref, dinv_ref, b_ref, w_ref, ys_ref):
        agg = p_ref[0] + p_ref[1] + xs_ref[...]
        dinv = dinv_ref[...]
        h1 = jnp.tanh(agg * dinv + b_ref[...])
        ys = jnp.dot(h1, w_ref[...], preferred_element_type=jnp.float32)
        ys_ref[...] = ys * dinv

    return pl.pallas_call(
        body,
        grid=(n // bn,),
        in_specs=[
            pl.BlockSpec((_NC, bn, d_hid), lambda i: (0, i, 0)),
            pl.BlockSpec((bn, d_hid), lambda i: (i, 0)),
            pl.BlockSpec((bn, 1), lambda i: (i, 0)),
            pl.BlockSpec((1, d_hid), lambda i: (0, 0)),
            pl.BlockSpec((d_hid, d_out), lambda i: (0, 0)),
        ],
        out_specs=pl.BlockSpec((bn, d_out), lambda i: (i, 0)),
        out_shape=jax.ShapeDtypeStruct((n, d_out), jnp.float32),
    )(p1, xs, dinv, b1, w2)


def _tc_last(p2, ys, dinv, b2, bn):
    n, d_out = ys.shape

    def body(p_ref, ys_ref, dinv_ref, b_ref, out_ref):
        agg = p_ref[0] + p_ref[1] + ys_ref[...]
        out_ref[...] = agg * dinv_ref[...] + b_ref[...]

    return pl.pallas_call(
        body,
        grid=(n // bn,),
        in_specs=[
            pl.BlockSpec((_NC, bn, d_out), lambda i: (0, i, 0)),
            pl.BlockSpec((bn, d_out), lambda i: (i, 0)),
            pl.BlockSpec((bn, 1), lambda i: (i, 0)),
            pl.BlockSpec((1, d_out), lambda i: (0, 0)),
        ],
        out_specs=pl.BlockSpec((bn, d_out), lambda i: (i, 0)),
        out_shape=jax.ShapeDtypeStruct((n, d_out), jnp.float32),
    )(p2, ys, dinv, b2)


def kernel(x, edge_index, W1, b1, W2, b2):
    n, d_in = x.shape
    d_hid = W1.shape[1]
    d_out = W2.shape[1]

    src3, dst3, nb = _pad_edges(edge_index[0], edge_index[1], n)
    rpt = -(-(n + 8) // (_NS * 16)) * 16
    bn = 2000 if n % 2000 == 0 else 8

    t = _tc_matmul(x, W1, bn)
    d0, d1 = _make_deg_kernel(n, nb, rpt)(dst3)
    dp2 = jnp.stack([d0, d1]).reshape(_NC, n, 1)
    xs, dinv = _tc_scale(dp2, t, bn)

    z1 = jnp.zeros((_agg_cw(n, rpt, d_hid), d_hid), jnp.float32)
    p1 = _make_agg_kernel(n, nb, _RING1, rpt, d_hid)(xs, src3, dst3, z1)

    ys = _tc_mid(p1, xs, dinv, b1.reshape(1, d_hid), W2, bn)

    z2 = jnp.zeros((_agg_cw(n, rpt, d_out), d_out), jnp.float32)
    p2 = _make_agg_kernel(n, nb, _RING2, rpt, d_out)(ys, src3, dst3, z2)

    return _tc_last(p2, ys, dinv, b2.reshape(1, d_out), bn)


# speedup vs baseline: 1.0197x; 1.0197x over previous
import functools
import math
import jax
import jax.numpy as jnp
from jax import lax
from jax.experimental import pallas as pl
from jax.experimental.pallas import tpu as pltpu
from jax.experimental.pallas import tpu_sc as plsc

_NC = 2
_NS = 16
_NW = _NC * _NS
_RING1 = 2
_RING2 = 4

_MESH = plsc.VectorSubcoreMesh(
    core_axis_name="c", subcore_axis_name="s", num_cores=_NC, num_subcores=_NS
)


def _pad_edges(src, dst, n):
    e = src.shape[0]
    r = max(_RING1, _RING2, 8)
    nb = -(-(-(-e // (_NW * 128))) // r) * r
    e_pad = _NW * nb * 128
    pad = e_pad - e
    if pad:
        j = jnp.arange(pad, dtype=jnp.int32)
        src = jnp.concatenate([src, j % 16])
        dst = jnp.concatenate([dst, n + (j % 8)])
    return (
        (src.reshape(_NW, 2 * nb, 64), dst.reshape(_NW, 2 * nb, 64)),
        (src.reshape(_NW, nb, 128), dst.reshape(_NW, nb, 128)),
        2 * nb,
        nb,
    )


def _make_deg_kernel(n, nb, eb, rpt):
    n_pad = _NS * rpt
    last = n - (_NS - 1) * rpt

    @functools.partial(
        pl.kernel,
        out_type=[
            jax.ShapeDtypeStruct((n,), jnp.float32),
            jax.ShapeDtypeStruct((n,), jnp.float32),
        ],
        mesh=_MESH,
        scratch_types=[
            pltpu.VMEM((nb, eb), jnp.int32),
            pltpu.VMEM((eb,), jnp.float32),
            pltpu.VMEM((16,), jnp.float32),
            pltpu.VMEM((rpt,), jnp.float32),
            pltpu.VMEM_SHARED((n_pad,), jnp.float32),
            pltpu.SemaphoreType.DMA,
        ],
    )
    def deg_kernel(dst_hbm, out0_hbm, out1_hbm, dst_v, ones_v, z_v, wb_v,
                   hist_sh, sem):
        c = lax.axis_index("c")
        s = lax.axis_index("s")
        w = s * _NC + c
        start = pl.multiple_of(s * rpt, rpt)

        z_v[...] = jnp.zeros((16,), jnp.float32)
        for i in range(eb // 16):
            ones_v[pl.ds(i * 16, 16)] = jnp.ones((16,), jnp.float32)
        for k in range(rpt // 16):
            pltpu.sync_copy(z_v, hist_sh.at[pl.ds(start + k * 16, 16)])
        plsc.subcore_barrier()

        pltpu.sync_copy(dst_hbm.at[w], dst_v)

        def step(j, carry):
            pltpu.sync_copy(ones_v, hist_sh.at[dst_v.at[j]], add=True)
            return carry

        lax.fori_loop(0, nb, step, 0)
        plsc.subcore_barrier()

        for cc, out_hbm in ((0, out0_hbm), (1, out1_hbm)):

            @pl.when(jnp.logical_and(c == cc, s < _NS - 1))
            def _():
                pltpu.sync_copy(hist_sh.at[pl.ds(start, rpt)], wb_v)
                pltpu.sync_copy(wb_v, out_hbm.at[pl.ds(start, rpt)])

            @pl.when(jnp.logical_and(c == cc, s == _NS - 1))
            def _():
                pltpu.sync_copy(
                    hist_sh.at[pl.ds((_NS - 1) * rpt, last)], wb_v.at[pl.ds(0, last)]
                )
                pltpu.sync_copy(
                    wb_v.at[pl.ds(0, last)], out_hbm.at[pl.ds((_NS - 1) * rpt, last)]
                )

    return deg_kernel


def _agg_cw(n, rpt, d):
    cw = math.gcd(rpt, n - (_NS - 1) * rpt)
    while cw * d * 4 > 20 * 1024 and cw % 16 == 0:
        cw //= 2
    return cw


def _make_agg_kernel(n, nb, eb, ring, rpt, d):
    n_pad = _NS * rpt
    last = n - (_NS - 1) * rpt
    cw = _agg_cw(n, rpt, d)

    @functools.partial(
        pl.kernel,
        out_type=jax.ShapeDtypeStruct((_NC, n, d), jnp.float32),
        mesh=_MESH,
        compiler_params=pltpu.CompilerParams(use_tc_tiling_on_sc=False),
        scratch_types=[
            pltpu.VMEM((nb, eb), jnp.int32),
            pltpu.VMEM((nb, eb), jnp.int32),
            [pltpu.VMEM((eb, d), jnp.float32)] * ring,
            pltpu.VMEM((cw, d), jnp.float32),
            pltpu.VMEM_SHARED((n_pad, d), jnp.float32),
            [pltpu.SemaphoreType.DMA] * ring,
        ],
    )
    def agg_kernel(rows_hbm, src_hbm, dst_hbm, z_hbm, out_hbm,
                   src_v, dst_v, msgs, wb_v, acc_sh, sems):
        c = lax.axis_index("c")
        s = lax.axis_index("s")
        w = s * _NC + c
        start = pl.multiple_of(s * rpt, rpt)

        pltpu.sync_copy(z_hbm, wb_v)
        for k in range(rpt // cw):
            pltpu.sync_copy(wb_v, acc_sh.at[pl.ds(start + k * cw, cw)])
        plsc.subcore_barrier()

        pltpu.sync_copy(src_hbm.at[w], src_v)
        pltpu.sync_copy(dst_hbm.at[w], dst_v)

        for b in range(ring - 1):
            pltpu.async_copy(rows_hbm.at[src_v.at[b]], msgs[b], sems[b])

        def step(i, carry):
            j = i * ring
            for b in range(ring):
                jj = j + b
                bn = (b + ring - 1) % ring

                @pl.when(jj + ring - 1 < nb)
                def _():
                    pltpu.async_copy(
                        rows_hbm.at[src_v.at[jj + ring - 1]], msgs[bn], sems[bn]
                    )

                pltpu.make_async_copy(
                    rows_hbm.at[src_v.at[jj]], msgs[b], sems[b]
                ).wait()
                pltpu.sync_copy(msgs[b], acc_sh.at[dst_v.at[jj]], add=True)
            return carry

        lax.fori_loop(0, nb // ring, step, 0)
        plsc.subcore_barrier()

        @pl.when(s < _NS - 1)
        def _():
            for t in range(rpt // cw):
                pltpu.sync_copy(acc_sh.at[pl.ds(start + t * cw, cw)], wb_v)
                pltpu.sync_copy(wb_v, out_hbm.at[c, pl.ds(start + t * cw, cw)])

        @pl.when(s == _NS - 1)
        def _():
            for t in range(last // cw):
                off = (_NS - 1) * rpt + t * cw
                pltpu.sync_copy(acc_sh.at[pl.ds(off, cw)], wb_v)
                pltpu.sync_copy(wb_v, out_hbm.at[c, pl.ds(off, cw)])

    return agg_kernel


def _tc_matmul(x, w1, bn):
    n, d_in = x.shape
    d_hid = w1.shape[1]

    def body(x_ref, w_ref, t_ref):
        t_ref[...] = jnp.dot(
            x_ref[...], w_ref[...], preferred_element_type=jnp.float32
        )

    return pl.pallas_call(
        body,
        grid=(n // bn,),
        in_specs=[
            pl.BlockSpec((bn, d_in), lambda i: (i, 0)),
            pl.BlockSpec((d_in, d_hid), lambda i: (0, 0)),
        ],
        out_specs=pl.BlockSpec((bn, d_hid), lambda i: (i, 0)),
        out_shape=jax.ShapeDtypeStruct((n, d_hid), jnp.float32),
    )(x, w1)


def _tc_scale(d0, d1, t, bn):
    n, d_hid = t.shape

    def body(d0_ref, d1_ref, t_ref, xs_ref, dinv_ref):
        deg = d0_ref[...] + d1_ref[...] + 1.0
        dinv = lax.rsqrt(jnp.maximum(deg, 1.0))
        xs_ref[...] = t_ref[...] * dinv
        dinv_ref[...] = dinv

    return pl.pallas_call(
        body,
        grid=(n // bn,),
        in_specs=[
            pl.BlockSpec((bn, 1), lambda i: (i, 0)),
            pl.BlockSpec((bn, 1), lambda i: (i, 0)),
            pl.BlockSpec((bn, d_hid), lambda i: (i, 0)),
        ],
        out_specs=[
            pl.BlockSpec((bn, d_hid), lambda i: (i, 0)),
            pl.BlockSpec((bn, 1), lambda i: (i, 0)),
        ],
        out_shape=[
            jax.ShapeDtypeStruct((n, d_hid), jnp.float32),
            jax.ShapeDtypeStruct((n, 1), jnp.float32),
        ],
    )(d0, d1, t)


def _tc_mid(p1, xs, dinv, b1, w2, bn):
    n, d_hid = xs.shape
    d_out = w2.shape[1]

    def body(p_ref, xs_ref, dinv_ref, b_ref, w_ref, ys_ref):
        agg = p_ref[0] + p_ref[1] + xs_ref[...]
        dinv = dinv_ref[...]
        h1 = jnp.tanh(agg * dinv + b_ref[...])
        ys = jnp.dot(h1, w_ref[...], preferred_element_type=jnp.float32)
        ys_ref[...] = ys * dinv

    return pl.pallas_call(
        body,
        grid=(n // bn,),
        in_specs=[
            pl.BlockSpec((_NC, bn, d_hid), lambda i: (0, i, 0)),
            pl.BlockSpec((bn, d_hid), lambda i: (i, 0)),
            pl.BlockSpec((bn, 1), lambda i: (i, 0)),
            pl.BlockSpec((1, d_hid), lambda i: (0, 0)),
            pl.BlockSpec((d_hid, d_out), lambda i: (0, 0)),
        ],
        out_specs=pl.BlockSpec((bn, d_out), lambda i: (i, 0)),
        out_shape=jax.ShapeDtypeStruct((n, d_out), jnp.float32),
    )(p1, xs, dinv, b1, w2)


def _tc_last(p2, ys, dinv, b2, bn):
    n, d_out = ys.shape

    def body(p_ref, ys_ref, dinv_ref, b_ref, out_ref):
        agg = p_ref[0] + p_ref[1] + ys_ref[...]
        out_ref[...] = agg * dinv_ref[...] + b_ref[...]

    return pl.pallas_call(
        body,
        grid=(n // bn,),
        in_specs=[
            pl.BlockSpec((_NC, bn, d_out), lambda i: (0, i, 0)),
            pl.BlockSpec((bn, d_out), lambda i: (i, 0)),
            pl.BlockSpec((bn, 1), lambda i: (i, 0)),
            pl.BlockSpec((1, d_out), lambda i: (0, 0)),
        ],
        out_specs=pl.BlockSpec((bn, d_out), lambda i: (i, 0)),
        out_shape=jax.ShapeDtypeStruct((n, d_out), jnp.float32),
    )(p2, ys, dinv, b2)


def kernel(x, edge_index, W1, b1, W2, b2):
    n, d_in = x.shape
    d_hid = W1.shape[1]
    d_out = W2.shape[1]

    (src64, dst64), (src128, dst128), nb64, nb128 = _pad_edges(
        edge_index[0], edge_index[1], n
    )
    rpt = -(-(n + 8) // (_NS * 16)) * 16
    bn = 2000 if n % 2000 == 0 else 8

    t = _tc_matmul(x, W1, bn)
    d0, d1 = _make_deg_kernel(n, nb128, 128, rpt)(dst128)
    xs, dinv = _tc_scale(d0.reshape(n, 1), d1.reshape(n, 1), t, bn)

    z1 = jnp.zeros((_agg_cw(n, rpt, d_hid), d_hid), jnp.float32)
    p1 = _make_agg_kernel(n, nb64, 64, _RING1, rpt, d_hid)(xs, src64, dst64, z1)

    ys = _tc_mid(p1, xs, dinv, b1.reshape(1, d_hid), W2, bn)

    z2 = jnp.zeros((_agg_cw(n, rpt, d_out), d_out), jnp.float32)
    p2 = _make_agg_kernel(n, nb128, 128, _RING2, rpt, d_out)(ys, src128, dst128, z2)

    return _tc_last(p2, ys, dinv, b2.reshape(1, d_out), bn)
